# baseline (speedup 1.0000x reference)
import functools

import jax
import jax.numpy as jnp
from jax import lax
from jax.experimental import pallas as pl
from jax.experimental.pallas import tpu as pltpu
from jax.experimental.pallas import tpu_sc as plsc

_B, _S = 8, 32
_NCH, _NFEAT, _D = 4, 32, 2048
_NLEV = 100
_R = _B * _S * _NCH

_NC, _NS, _L = 2, 16, 16
_NW = _NC * _NS
_WC = 128
_WR = _R // 2
_NQ = _WC // _L
_FG = 4
_NG = _NFEAT // _FG


def _prep_body(samples_ref, vw_ref, fw_ref, cw_ref,
               av_ref, vwc_ref, off_ref):
    s = samples_ref[...]
    idxf = jnp.round((s + 1.0) / 2.0 * 99.0)
    av_ref[...] = jnp.clip(idxf, 0.0, 99.0).astype(jnp.int32)

    csum = jnp.sum(cw_ref[...], axis=0, keepdims=True)
    vwc_ref[...] = vw_ref[...] * csum

    fw = fw_ref[...]
    off = jnp.where(fw < 0.0, _NLEV, 0)
    off_ref[...] = jnp.where(fw == 0.0, 2 * _NLEV, off)


@functools.partial(
    pl.kernel,
    mesh=plsc.VectorSubcoreMesh(core_axis_name="c", subcore_axis_name="s"),
    out_type=jax.ShapeDtypeStruct((_R, _D), jnp.float32),
    scratch_types=[
        pltpu.VMEM((_WR * _NFEAT,), jnp.int32),
        pltpu.VMEM((3 * _NLEV, _WC), jnp.float32),
        pltpu.VMEM((_NFEAT, _WC), jnp.int32),
        pltpu.VMEM((_WR, _WC), jnp.float32),
    ],
    compiler_params=pltpu.CompilerParams(needs_layout_passes=False),
)
def _sc_s4(av_hbm, vwc_hbm, off_hbm, s4_hbm, av_v, tab_v, off_v, s4_v):
    wid = lax.axis_index("s") * _NC + lax.axis_index("c")
    ctile = wid // 2
    rhalf = wid % 2
    cols = pl.ds(ctile * _WC, _WC)
    pltpu.sync_copy(av_hbm.at[pl.ds(rhalf * _WR * _NFEAT, _WR * _NFEAT)],
                    av_v)
    pltpu.sync_copy(vwc_hbm.at[:, cols], tab_v.at[pl.ds(0, _NLEV)])
    pltpu.sync_copy(off_hbm.at[:, cols], off_v)

    zero = jnp.zeros((_L,), jnp.float32)

    def neg_body(i, _):
        for q in range(_NQ):
            v = tab_v[i, pl.ds(q * _L, _L)]
            tab_v[_NLEV + i, pl.ds(q * _L, _L)] = -v
            tab_v[2 * _NLEV + i, pl.ds(q * _L, _L)] = zero
        return 0

    lax.fori_loop(0, _NLEV, neg_body, 0)

    colqs = [lax.iota(jnp.int32, _L) + q * _L for q in range(_NQ)]

    for g in range(_NG):
        offs = [[off_v[g * _FG + f, pl.ds(q * _L, _L)] for q in range(_NQ)]
                for f in range(_FG)]

        def r_body(r, _, g=g, offs=offs):
            accs = [jnp.zeros((_L,), jnp.float32) for _ in range(_NQ)]
            for f in range(_FG):
                ai = plsc.load_gather(
                    av_v, [jnp.full((_L,), r * _NFEAT + g * _FG + f,
                                    dtype=jnp.int32)])
                for q in range(_NQ):
                    accs[q] = accs[q] + plsc.load_gather(
                        tab_v, [ai + offs[f][q], colqs[q]])
            for q in range(_NQ):
                if g == 0:
                    s4_v[r, pl.ds(q * _L, _L)] = accs[q]
                else:
                    plsc.addupdate(s4_v.at[r, pl.ds(q * _L, _L)], accs[q])
            return 0

        lax.fori_loop(0, _WR, r_body, 0)

    pltpu.sync_copy(s4_v, s4_hbm.at[pl.ds(rhalf * _WR, _WR), cols])


def kernel(samples, component_weight, feat_weight, value_weight):
    samples_r = samples.reshape(_R, _NFEAT)
    av, vwc, off = pl.pallas_call(
        _prep_body,
        out_shape=(
            jax.ShapeDtypeStruct((_R, _NFEAT), jnp.int32),
            jax.ShapeDtypeStruct((_NLEV, _D), jnp.float32),
            jax.ShapeDtypeStruct((_NFEAT, _D), jnp.int32),
        ),
    )(samples_r, value_weight, feat_weight, component_weight)

    s4 = _sc_s4(av.reshape(-1), vwc, off).reshape(_B, _S, _NCH, _D)

    t_interp = jax.nn.sigmoid(s4[:, :, 2, :] + s4[:, :, 3, :])
    h = s4[:, :, 0, :] * (1.0 - t_interp) + t_interp * s4[:, :, 1, :]
    h = jnp.roll(h, shift=1, axis=0)
    h = h.at[0].set(jnp.zeros_like(h[0]))
    s4 = s4 + h[:, :, None, :]
    return jnp.sign(jnp.sum(s4.reshape(_B, _S, -1), axis=1))


# speedup vs baseline: 1.0150x; 1.0150x over previous
import functools

import jax
import jax.numpy as jnp
from jax import lax
from jax.experimental import pallas as pl
from jax.experimental.pallas import tpu as pltpu
from jax.experimental.pallas import tpu_sc as plsc

_B, _S = 8, 32
_NCH, _NFEAT, _D = 4, 32, 2048
_NLEV = 100
_R = _B * _S * _NCH

_NC, _NS, _L = 2, 16, 16
_NW = _NC * _NS
_WC = 128
_WR = _R // 2
_NQ = _WC // _L
_FG = 4
_NG = _NFEAT // _FG


def _prep_body(samples_ref, vw_ref, fw_ref, cw_ref,
               av_ref, vwc_ref, off_ref):
    s = samples_ref[...]
    idxf = jnp.round((s + 1.0) / 2.0 * 99.0)
    av_ref[...] = jnp.clip(idxf, 0.0, 99.0).astype(jnp.int32)

    csum = jnp.sum(cw_ref[...], axis=0, keepdims=True)
    vwc_ref[...] = vw_ref[...] * csum

    fw = fw_ref[...]
    off = jnp.where(fw < 0.0, _NLEV, 0)
    off_ref[...] = jnp.where(fw == 0.0, 2 * _NLEV, off)


@functools.partial(
    pl.kernel,
    mesh=plsc.VectorSubcoreMesh(core_axis_name="c", subcore_axis_name="s"),
    out_type=jax.ShapeDtypeStruct((_R, _D), jnp.float32),
    scratch_types=[
        pltpu.VMEM((_WR * _NFEAT,), jnp.int32),
        pltpu.VMEM((3 * _NLEV, _WC), jnp.float32),
        pltpu.VMEM((_NFEAT, _WC), jnp.int32),
        pltpu.VMEM((_WR, _WC), jnp.float32),
    ],
    compiler_params=pltpu.CompilerParams(needs_layout_passes=False),
)
def _sc_s4(av_hbm, vwc_hbm, off_hbm, s4_hbm, av_v, tab_v, off_v, s4_v):
    wid = lax.axis_index("s") * _NC + lax.axis_index("c")
    ctile = wid // 2
    rhalf = wid % 2
    cols = pl.ds(ctile * _WC, _WC)
    pltpu.sync_copy(av_hbm.at[pl.ds(rhalf * _WR * _NFEAT, _WR * _NFEAT)],
                    av_v)
    pltpu.sync_copy(vwc_hbm.at[:, cols], tab_v.at[pl.ds(0, _NLEV)])
    pltpu.sync_copy(off_hbm.at[:, cols], off_v)

    zero = jnp.zeros((_L,), jnp.float32)

    def neg_body(i, _):
        for q in range(_NQ):
            v = tab_v[i, pl.ds(q * _L, _L)]
            tab_v[_NLEV + i, pl.ds(q * _L, _L)] = -v
            tab_v[2 * _NLEV + i, pl.ds(q * _L, _L)] = zero
        return 0

    lax.fori_loop(0, _NLEV, neg_body, 0)

    colqs = [lax.iota(jnp.int32, _L) + q * _L for q in range(_NQ)]

    for g in range(_NG):
        offs = [[off_v[g * _FG + f, pl.ds(q * _L, _L)] for q in range(_NQ)]
                for f in range(_FG)]

        def r_body(r, _, g=g, offs=offs):
            if g == 0:
                accs = [jnp.zeros((_L,), jnp.float32) for _ in range(_NQ)]
            else:
                accs = [s4_v[r, pl.ds(q * _L, _L)] for q in range(_NQ)]
            for f in range(_FG):
                ai = plsc.load_gather(
                    av_v, [jnp.full((_L,), r * _NFEAT + g * _FG + f,
                                    dtype=jnp.int32)])
                for q in range(_NQ):
                    accs[q] = accs[q] + plsc.load_gather(
                        tab_v, [ai + offs[f][q], colqs[q]])
            for q in range(_NQ):
                s4_v[r, pl.ds(q * _L, _L)] = accs[q]
            return 0

        lax.fori_loop(0, _WR, r_body, 0)

    pltpu.sync_copy(s4_v, s4_hbm.at[pl.ds(rhalf * _WR, _WR), cols])


def kernel(samples, component_weight, feat_weight, value_weight):
    samples_r = samples.reshape(_R, _NFEAT)
    av, vwc, off = pl.pallas_call(
        _prep_body,
        out_shape=(
            jax.ShapeDtypeStruct((_R, _NFEAT), jnp.int32),
            jax.ShapeDtypeStruct((_NLEV, _D), jnp.float32),
            jax.ShapeDtypeStruct((_NFEAT, _D), jnp.int32),
        ),
    )(samples_r, value_weight, feat_weight, component_weight)

    s4 = _sc_s4(av.reshape(-1), vwc, off).reshape(_B, _S, _NCH, _D)

    t_interp = jax.nn.sigmoid(s4[:, :, 2, :] + s4[:, :, 3, :])
    h = s4[:, :, 0, :] * (1.0 - t_interp) + t_interp * s4[:, :, 1, :]
    h = jnp.roll(h, shift=1, axis=0)
    h = h.at[0].set(jnp.zeros_like(h[0]))
    s4 = s4 + h[:, :, None, :]
    return jnp.sign(jnp.sum(s4.reshape(_B, _S, -1), axis=1))
